# Initial kernel scaffold; baseline (speedup 1.0000x reference)
import jax
import jax.numpy as jnp
from jax.experimental import pallas as pl


def kernel(M, perm):
    raise NotImplementedError("write your pallas kernel here")


# speedup vs baseline: 7.0029x; 7.0029x over previous
import jax
import jax.numpy as jnp
from jax.experimental import pallas as pl

D = 1024


def _permute_body(perm_col_ref, perm_row_ref, m_ref, out_ref):
    col = jax.lax.broadcasted_iota(jnp.int32, (D, D), 1)
    row = jax.lax.broadcasted_iota(jnp.int32, (D, D), 0)
    p = (perm_col_ref[...] == col).astype(jnp.float32)
    pt = (perm_row_ref[...] == row).astype(jnp.float32)
    r = jnp.dot(p, m_ref[...], preferred_element_type=jnp.float32)
    out_ref[...] = jnp.dot(r, pt, preferred_element_type=jnp.float32)


def kernel(M, perm):
    perm_col = perm.reshape(D, 1).astype(jnp.int32)
    perm_row = perm.reshape(1, D).astype(jnp.int32)
    dag = pl.pallas_call(
        _permute_body,
        out_shape=jax.ShapeDtypeStruct((D, D), jnp.float32),
    )(perm_col, perm_row, M)
    probs = jnp.ones((1,), dtype=jnp.float32)
    reg = jnp.zeros(())
    return (probs, dag[None, ...], reg)
